# baseline (speedup 1.0000x reference)
import functools

import jax
import jax.numpy as jnp
from jax import lax
from jax.experimental import pallas as pl
from jax.experimental.pallas import tpu as pltpu
from jax.experimental.pallas import tpu_sc as plsc

BATCH = 4096
HIST = 50
VOCAB = 16384
EMB_DIM = 64
OUT_DIM = EMB_DIM + 1
ID_MASK = (1 << 14) - 1
BAD_MASK = (1 << 14) | (1 << 15)

NC = 2
NS = 16
NW = NC * NS
L = 16

B_TOTAL = BATCH * HIST
PER_B = BATCH // NW
CB = 32
NCH = PER_B // CB
CHUNK = CB * HIST
GPAD = 56
IDX_OFFS = (0, 16, 32, 40)


def _sc_body(qc_hbm, emb_hbm, feat_hbm, val_hbm, qc_v, idx_v, val_v, buf_v, sem):
    wid = lax.axis_index("s") * NC + lax.axis_index("c")
    bbase = wid * PER_B

    def chunk_body(c, carry):
        b0 = bbase + c * CB
        cbase = b0 * HIST
        pltpu.sync_copy(qc_hbm.at[pl.ds(cbase, CHUNK)], qc_v.at[pl.ds(0, CHUNK)])

        def val_body(i, carry):
            q = qc_v[pl.ds(i * L, L)]
            val_v[pl.ds(i * L, L)] = jnp.where((q & BAD_MASK) == 0, 1.0, 0.0)
            return carry

        lax.fori_loop(0, CHUNK // L, val_body, 0)

        lane = lax.iota(jnp.int32, L)

        def idx_body(n, carry):
            b = n // len(IDX_OFFS)
            s = n % len(IDX_OFFS)
            o = jnp.where(s == 3, IDX_OFFS[3], s * L)
            q = qc_v[pl.ds(b * HIST + o, L)]
            idx_v[b, pl.ds(o, L)] = jnp.where(
                o + lane < HIST, q & ID_MASK, 0
            )
            return carry

        lax.fori_loop(0, CB * len(IDX_OFFS), idx_body, 0)

        handles = [
            pltpu.async_copy(emb_hbm.at[idx_v.at[b]], buf_v.at[b], sem)
            for b in range(CB)
        ]
        for h in handles:
            h.wait()

        pltpu.sync_copy(buf_v, feat_hbm.at[pl.ds(b0, CB)])
        pltpu.sync_copy(val_v, val_hbm.at[pl.ds(cbase, CHUNK)])
        return carry

    lax.fori_loop(0, NCH, chunk_body, 0)


_call = functools.partial(
    pl.kernel,
    out_type=(
        jax.ShapeDtypeStruct((BATCH, GPAD, EMB_DIM), jnp.float32),
        jax.ShapeDtypeStruct((B_TOTAL,), jnp.float32),
    ),
    mesh=plsc.VectorSubcoreMesh(core_axis_name="c", subcore_axis_name="s"),
    scratch_types=[
        pltpu.VMEM((CHUNK + 64,), jnp.int32),
        pltpu.VMEM((CB, GPAD), jnp.int32),
        pltpu.VMEM((CHUNK,), jnp.float32),
        pltpu.VMEM((CB, GPAD, EMB_DIM), jnp.float32),
        pltpu.SemaphoreType.DMA,
    ],
    compiler_params=pltpu.CompilerParams(use_tc_tiling_on_sc=False),
)(_sc_body)


@jax.jit
def kernel(qc_flags, emb):
    qc_flat = qc_flags.astype(jnp.int32).reshape(B_TOTAL)
    feat, valid = _call(qc_flat, emb)
    return jnp.concatenate(
        [feat[:, :HIST, :], valid.reshape(BATCH, HIST, 1)], axis=-1
    )


# speedup vs baseline: 3.9984x; 3.9984x over previous
import functools

import jax
import jax.numpy as jnp
from jax import lax
from jax.experimental import pallas as pl
from jax.experimental.pallas import tpu as pltpu
from jax.experimental.pallas import tpu_sc as plsc

BATCH = 4096
HIST = 50
VOCAB = 16384
EMB_DIM = 64
OUT_DIM = EMB_DIM + 1
ID_MASK = (1 << 14) - 1
BAD_MASK = (1 << 14) | (1 << 15)

NC = 2
NS = 16
NW = NC * NS
L = 16

B_TOTAL = BATCH * HIST
PER_W = B_TOTAL // NW
CHUNK = 640
NCH = PER_W // CHUNK
GROWS = 128
NG = CHUNK // GROWS
VPG = GROWS // L


def _sc_body(
    qc_hbm, emb_hbm, feat_hbm, val_hbm,
    qc_v0, qc_v1, idx_v0, idx_v1, val_v0, val_v1, buf_v0, buf_v1,
    sem_g, sem_w,
):
    wid = lax.axis_index("s") * NC + lax.axis_index("c")
    base = wid * PER_W
    qc_b = (qc_v0, qc_v1)
    idx_b = (idx_v0, idx_v1)
    val_b = (val_v0, val_v1)
    buf_b = (buf_v0, buf_v1)

    pending = {}
    for c in range(NCH):
        p = c % 2
        qc_v, idx_v, val_v, buf_v = qc_b[p], idx_b[p], val_b[p], buf_b[p]
        if c >= 2:
            for h in pending.pop(c - 2):
                h.wait()
        cbase = base + c * CHUNK
        pltpu.sync_copy(qc_hbm.at[pl.ds(cbase, CHUNK)], qc_v)

        def idx_body(i, carry, qc_v=qc_v, idx_v=idx_v, val_v=val_v):
            j = i // VPG
            t = i % VPG
            q = qc_v[pl.ds(j * GROWS + t * L, L)]
            idx_v[j, pl.ds(t * L, L)] = q & ID_MASK
            val_v[pl.ds(j * GROWS + t * L, L)] = jnp.where(
                (q & BAD_MASK) == 0, 1.0, 0.0
            )
            return carry

        lax.fori_loop(0, NG * VPG, idx_body, 0)

        gathers = [
            pltpu.async_copy(
                emb_hbm.at[idx_v.at[j]],
                buf_v.at[pl.ds(j * GROWS, GROWS)],
                sem_g,
            )
            for j in range(NG)
        ]
        for h in gathers:
            h.wait()

        pending[c] = [
            pltpu.async_copy(buf_v, feat_hbm.at[pl.ds(cbase, CHUNK)], sem_w),
            pltpu.async_copy(val_v, val_hbm.at[pl.ds(cbase, CHUNK)], sem_w),
        ]

    for c in sorted(pending):
        for h in pending[c]:
            h.wait()


_call = functools.partial(
    pl.kernel,
    out_type=(
        jax.ShapeDtypeStruct((B_TOTAL, EMB_DIM), jnp.float32),
        jax.ShapeDtypeStruct((B_TOTAL,), jnp.float32),
    ),
    mesh=plsc.VectorSubcoreMesh(core_axis_name="c", subcore_axis_name="s"),
    scratch_types=[
        pltpu.VMEM((CHUNK,), jnp.int32),
        pltpu.VMEM((CHUNK,), jnp.int32),
        pltpu.VMEM((NG, GROWS), jnp.int32),
        pltpu.VMEM((NG, GROWS), jnp.int32),
        pltpu.VMEM((CHUNK,), jnp.float32),
        pltpu.VMEM((CHUNK,), jnp.float32),
        pltpu.VMEM((CHUNK, EMB_DIM), jnp.float32),
        pltpu.VMEM((CHUNK, EMB_DIM), jnp.float32),
        pltpu.SemaphoreType.DMA,
        pltpu.SemaphoreType.DMA,
    ],
    compiler_params=pltpu.CompilerParams(use_tc_tiling_on_sc=False),
)(_sc_body)


@jax.jit
def kernel(qc_flags, emb):
    qc_flat = qc_flags.astype(jnp.int32).reshape(B_TOTAL)
    feat, valid = _call(qc_flat, emb)
    return jnp.concatenate(
        [
            feat.reshape(BATCH, HIST, EMB_DIM),
            valid.reshape(BATCH, HIST, 1),
        ],
        axis=-1,
    )


# speedup vs baseline: 4.0026x; 1.0011x over previous
import functools

import jax
import jax.numpy as jnp
from jax import lax
from jax.experimental import pallas as pl
from jax.experimental.pallas import tpu as pltpu
from jax.experimental.pallas import tpu_sc as plsc

BATCH = 4096
HIST = 50
VOCAB = 16384
EMB_DIM = 64
OUT_DIM = EMB_DIM + 1
ID_MASK = (1 << 14) - 1
BAD_MASK = (1 << 14) | (1 << 15)

NC = 2
NS = 16
NW = NC * NS
L = 16

B_TOTAL = BATCH * HIST
PER_W = B_TOTAL // NW
CHUNK = 640
NCH = PER_W // CHUNK
GROWS = 128
NG = CHUNK // GROWS
VPG = GROWS // L


def _sc_body(
    qc_hbm, emb_hbm, feat_hbm, val_hbm,
    qc_v0, qc_v1, idx_v0, idx_v1, val_v0, val_v1, buf_v0, buf_v1,
    sem_g, sem_w,
):
    wid = lax.axis_index("s") * NC + lax.axis_index("c")
    base = wid * PER_W
    qc_b = (qc_v0, qc_v1)
    idx_b = (idx_v0, idx_v1)
    val_b = (val_v0, val_v1)
    buf_b = (buf_v0, buf_v1)

    pending = {}
    for c in range(NCH):
        p = c % 2
        qc_v, idx_v, val_v, buf_v = qc_b[p], idx_b[p], val_b[p], buf_b[p]
        if c >= 2:
            for h in pending.pop(c - 2):
                h.wait()
        cbase = base + c * CHUNK
        pltpu.sync_copy(qc_hbm.at[pl.ds(cbase, CHUNK)], qc_v)

        def idx_body(i, carry, qc_v=qc_v, idx_v=idx_v, val_v=val_v):
            j = i // VPG
            t = i % VPG
            q = qc_v[pl.ds(j * GROWS + t * L, L)]
            idx_v[j, pl.ds(t * L, L)] = q & ID_MASK
            val_v[pl.ds(j * GROWS + t * L, L)] = jnp.where(
                (q & BAD_MASK) == 0, 1.0, 0.0
            )
            return carry

        lax.fori_loop(0, NG * VPG, idx_body, 0)

        gathers = [
            pltpu.async_copy(
                emb_hbm.at[idx_v.at[j]],
                buf_v.at[pl.ds(j * GROWS, GROWS)],
                sem_g,
            )
            for j in range(NG)
        ]
        for h in gathers:
            h.wait()

        pending[c] = [
            pltpu.async_copy(buf_v, feat_hbm.at[pl.ds(cbase, CHUNK)], sem_w),
            pltpu.async_copy(val_v, val_hbm.at[pl.ds(cbase, CHUNK)], sem_w),
        ]

    for c in sorted(pending):
        for h in pending[c]:
            h.wait()


_call = functools.partial(
    pl.kernel,
    out_type=(
        jax.ShapeDtypeStruct((B_TOTAL, EMB_DIM), jnp.float32),
        jax.ShapeDtypeStruct((B_TOTAL,), jnp.float32),
    ),
    mesh=plsc.VectorSubcoreMesh(core_axis_name="c", subcore_axis_name="s"),
    scratch_types=[
        pltpu.VMEM((CHUNK,), jnp.int32),
        pltpu.VMEM((CHUNK,), jnp.int32),
        pltpu.VMEM((NG, GROWS), jnp.int32),
        pltpu.VMEM((NG, GROWS), jnp.int32),
        pltpu.VMEM((CHUNK,), jnp.float32),
        pltpu.VMEM((CHUNK,), jnp.float32),
        pltpu.VMEM((CHUNK, EMB_DIM), jnp.float32),
        pltpu.VMEM((CHUNK, EMB_DIM), jnp.float32),
        pltpu.SemaphoreType.DMA,
        pltpu.SemaphoreType.DMA,
    ],
    compiler_params=pltpu.CompilerParams(use_tc_tiling_on_sc=False),
)(_sc_body)


@jax.jit
def kernel(qc_flags, emb):
    qc_flat = qc_flags.astype(jnp.int32).reshape(B_TOTAL)
    feat, valid = _call(qc_flat, emb)
    out2d = jnp.concatenate([feat, valid[:, None]], axis=-1)
    return out2d.reshape(BATCH, HIST, OUT_DIM)
